# Initial kernel scaffold; baseline (speedup 1.0000x reference)
import jax
import jax.numpy as jnp
from jax.experimental import pallas as pl


def kernel(x, edge_index, W_emb, b_emb, W1, b1, W2, b2, W3, b3, W_fc1, b_fc1, W_fc2, b_fc2):
    raise NotImplementedError("write your pallas kernel here")


# speedup vs baseline: 6.6402x; 6.6402x over previous
import functools

import jax
import jax.numpy as jnp
from jax import lax
from jax.experimental import pallas as pl
from jax.experimental.pallas import tpu as pltpu
from jax.experimental.pallas import tpu_sc as plsc

N = 10000
H = 128
OUT = 64
E = 320000

NTILES = 32
NSUB = 16
NP = 10240
STRIPE = NP // NSUB
E_PAD = 327680
EP = E_PAD // NTILES
CHUNK = 128
NCHUNK = EP // CHUNK
DEGW = 16

ROWB = 1000
NBLK = N // ROWB

_mesh = plsc.VectorSubcoreMesh(core_axis_name="c", subcore_axis_name="s")


@functools.partial(
    pl.kernel,
    out_type=jax.ShapeDtypeStruct((2, NP, DEGW), jnp.float32),
    mesh=_mesh,
    scratch_types=[
        pltpu.VMEM((CHUNK,), jnp.int32),
        pltpu.VMEM((CHUNK, DEGW), jnp.float32),
        pltpu.VMEM((16, DEGW), jnp.float32),
        pltpu.VMEM_SHARED((NP, DEGW), jnp.float32),
    ],
)
def _sc_degree(dst_hbm, out_hbm, didx, ones_v, zv, acc):
    c = lax.axis_index("c")
    s = lax.axis_index("s")
    wid = s * 2 + c
    one16 = jnp.ones((16,), jnp.float32)
    zero16 = jnp.zeros((16,), jnp.float32)
    for r in range(CHUNK):
        ones_v[r, :] = one16
    for r in range(16):
        zv[r, :] = zero16

    @pl.loop(0, STRIPE // 16)
    def _zero(j):
        pltpu.sync_copy(zv, acc.at[pl.ds(s * STRIPE + j * 16, 16)])

    plsc.subcore_barrier()

    @pl.loop(0, NCHUNK)
    def _scatter(j):
        base = wid * EP + j * CHUNK
        pltpu.sync_copy(dst_hbm.at[pl.ds(base, CHUNK)], didx)
        pltpu.sync_copy(ones_v, acc.at[didx], add=True)

    plsc.subcore_barrier()
    pltpu.sync_copy(acc.at[pl.ds(s * STRIPE, STRIPE)],
                    out_hbm.at[c, pl.ds(s * STRIPE, STRIPE)])


@functools.partial(
    pl.kernel,
    out_type=jax.ShapeDtypeStruct((2, NP, H), jnp.float32),
    mesh=_mesh,
    scratch_types=[
        pltpu.VMEM((CHUNK,), jnp.int32),
        pltpu.VMEM((CHUNK,), jnp.int32),
        pltpu.VMEM((CHUNK, H), jnp.float32),
        pltpu.VMEM((16, H), jnp.float32),
        pltpu.VMEM_SHARED((NP, H), jnp.float32),
        pltpu.SemaphoreType.DMA,
    ],
)
def _sc_scatter(msgs_hbm, src_hbm, dst_hbm, out_hbm, sidx, didx, rows, zv, acc, sem):
    c = lax.axis_index("c")
    s = lax.axis_index("s")
    wid = s * 2 + c
    zero16 = jnp.zeros((16,), jnp.float32)
    for r in range(16):
        for q in range(H // 16):
            zv[r, pl.ds(q * 16, 16)] = zero16

    @pl.loop(0, STRIPE // 16)
    def _zero(j):
        pltpu.sync_copy(zv, acc.at[pl.ds(s * STRIPE + j * 16, 16)])

    plsc.subcore_barrier()

    @pl.loop(0, NCHUNK)
    def _scatter(j):
        base = wid * EP + j * CHUNK
        pltpu.sync_copy(src_hbm.at[pl.ds(base, CHUNK)], sidx)
        pltpu.sync_copy(dst_hbm.at[pl.ds(base, CHUNK)], didx)
        pltpu.async_copy(msgs_hbm.at[sidx], rows, sem).wait()
        pltpu.sync_copy(rows, acc.at[didx], add=True)

    plsc.subcore_barrier()
    pltpu.sync_copy(acc.at[pl.ds(s * STRIPE, STRIPE)],
                    out_hbm.at[c, pl.ds(s * STRIPE, STRIPE)])


def _dinv_block(degA_ref, degB_ref):
    deg = degA_ref[0][:, 0:1] + degB_ref[0][:, 0:1] + 1.0
    return lax.rsqrt(deg)


def _tc1_body(x_ref, wemb_ref, bemb_ref, w1_ref, degA_ref, degB_ref, out_ref):
    dinv = _dinv_block(degA_ref, degB_ref)
    wc = jnp.dot(wemb_ref[...], w1_ref[...], preferred_element_type=jnp.float32)
    bc = jnp.dot(bemb_ref[...], w1_ref[...], preferred_element_type=jnp.float32)
    h = jnp.dot(x_ref[...], wc, preferred_element_type=jnp.float32) + bc
    out_ref[...] = dinv * h


def _tc_mid_body(accA_ref, accB_ref, msgs_ref, degA_ref, degB_ref, b_ref, w_ref,
                 out_ref):
    dinv = _dinv_block(degA_ref, degB_ref)
    pre = dinv * (accA_ref[0] + accB_ref[0] + msgs_ref[...]) + b_ref[...]
    h = jnp.maximum(pre, 0.0)
    out_ref[...] = dinv * jnp.dot(h, w_ref[...], preferred_element_type=jnp.float32)


def _tc_fin_body(accA_ref, accB_ref, msgs_ref, degA_ref, degB_ref, b_ref,
                 wfc1_ref, bfc1_ref, wfc2_ref, bfc2_ref, out_ref, sacc):
    i = pl.program_id(0)
    dinv = _dinv_block(degA_ref, degB_ref)
    pre = dinv * (accA_ref[0] + accB_ref[0] + msgs_ref[...]) + b_ref[...]
    h = jnp.maximum(pre, 0.0)
    part = jnp.sum(h, axis=0, keepdims=True)

    @pl.when(i == 0)
    def _init():
        sacc[...] = part

    @pl.when(i > 0)
    def _accum():
        sacc[...] = sacc[...] + part

    @pl.when(i == 0)
    def _zero_out():
        out_ref[...] = jnp.zeros((1, OUT), jnp.float32)

    @pl.when(i == NBLK - 1)
    def _head():
        g = sacc[...] * (1.0 / N)
        z = jnp.maximum(
            jnp.dot(g, wfc1_ref[...], preferred_element_type=jnp.float32)
            + bfc1_ref[...], 0.0)
        out_ref[...] = (jnp.dot(z, wfc2_ref[...],
                                preferred_element_type=jnp.float32)
                        + bfc2_ref[...])


_row_spec = pl.BlockSpec((ROWB, H), lambda i: (i, 0))
_w_spec = pl.BlockSpec((H, H), lambda i: (0, 0))
_b_spec = pl.BlockSpec((1, H), lambda i: (0, 0))
_degA_spec = pl.BlockSpec((1, ROWB, DEGW), lambda i: (0, i, 0))
_degB_spec = pl.BlockSpec((1, ROWB, DEGW), lambda i: (1, i, 0))
_accA_spec = pl.BlockSpec((1, ROWB, H), lambda i: (0, i, 0))
_accB_spec = pl.BlockSpec((1, ROWB, H), lambda i: (1, i, 0))

_tc1 = pl.pallas_call(
    _tc1_body,
    grid=(NBLK,),
    in_specs=[_row_spec, _w_spec, _b_spec, _w_spec, _degA_spec, _degB_spec],
    out_specs=_row_spec,
    out_shape=jax.ShapeDtypeStruct((N, H), jnp.float32),
)

_tc_mid = pl.pallas_call(
    _tc_mid_body,
    grid=(NBLK,),
    in_specs=[_accA_spec, _accB_spec, _row_spec, _degA_spec, _degB_spec,
              _b_spec, _w_spec],
    out_specs=_row_spec,
    out_shape=jax.ShapeDtypeStruct((N, H), jnp.float32),
)

_tc_fin = pl.pallas_call(
    _tc_fin_body,
    grid=(NBLK,),
    in_specs=[_accA_spec, _accB_spec, _row_spec, _degA_spec, _degB_spec,
              _b_spec, _w_spec, _b_spec,
              pl.BlockSpec((H, OUT), lambda i: (0, 0)),
              pl.BlockSpec((1, OUT), lambda i: (0, 0))],
    out_specs=pl.BlockSpec((1, OUT), lambda i: (0, 0)),
    out_shape=jax.ShapeDtypeStruct((1, OUT), jnp.float32),
    scratch_shapes=[pltpu.VMEM((1, H), jnp.float32)],
)


def kernel(x, edge_index, W_emb, b_emb, W1, b1, W2, b2, W3, b3,
           W_fc1, b_fc1, W_fc2, b_fc2):
    ei = edge_index.astype(jnp.int32)
    pad = E_PAD - E
    src = jnp.concatenate([ei[0], jnp.zeros((pad,), jnp.int32)])
    dst = jnp.concatenate(
        [ei[1], N + (jnp.arange(pad, dtype=jnp.int32) % (NP - N))])

    b_emb2 = b_emb.reshape(1, H)
    b1r = b1.reshape(1, H)
    b2r = b2.reshape(1, H)
    b3r = b3.reshape(1, H)
    bfc1r = b_fc1.reshape(1, H)
    bfc2r = b_fc2.reshape(1, OUT)

    deg2 = _sc_degree(dst)
    msgs1 = _tc1(x, W_emb, b_emb2, W1, deg2, deg2)
    acc1 = _sc_scatter(msgs1, src, dst)
    msgs2 = _tc_mid(acc1, acc1, msgs1, deg2, deg2, b1r, W2)
    acc2 = _sc_scatter(msgs2, src, dst)
    msgs3 = _tc_mid(acc2, acc2, msgs2, deg2, deg2, b2r, W3)
    acc3 = _sc_scatter(msgs3, src, dst)
    out = _tc_fin(acc3, acc3, msgs3, deg2, deg2, b3r, W_fc1, bfc1r,
                  W_fc2, bfc2r)
    return out
